# baseline (speedup 1.0000x reference)
import functools

import jax
import jax.numpy as jnp
from jax.experimental import pallas as pl
from jax.experimental.pallas import tpu as pltpu

_BQ = 256


def _warp_kernel(x1_ref, q2_ref, f1_ref, f1t_ref, out_ref, *, n1):
    x1 = x1_ref[0]
    f1 = f1_ref[0]
    f1t = f1t_ref[0]
    q = q2_ref[0]
    db = x1 + f1

    d = None
    for c in range(3):
        diff = q[:, c:c + 1] - db[c:c + 1, :]
        d = diff * diff if d is None else d + diff * diff

    iota = jax.lax.broadcasted_iota(jnp.int32, d.shape, 1)
    inf = jnp.float32(jnp.inf)

    invs = []
    dcur = d
    W = None
    for k in range(3):
        m = jnp.min(dcur, axis=1, keepdims=True)
        idx = jnp.min(jnp.where(dcur <= m, iota, n1), axis=1,
                      keepdims=True)
        oh = iota == idx
        inv = 1.0 / jnp.maximum(jnp.sqrt(m), 1e-10)
        invs.append(inv)
        W = jnp.where(oh, inv, 0.0) if k == 0 else jnp.where(oh, inv, W)
        if k < 2:
            dcur = jnp.where(oh, inf, dcur)

    rnorm = 1.0 / (invs[0] + invs[1] + invs[2])

    s = jax.lax.dot_general(W, f1t, (((1,), (0,)), ((), ())),
                            preferred_element_type=jnp.float32,
                            precision=jax.lax.Precision.HIGHEST)
    out_ref[0] = q - s * rnorm


@jax.jit
def kernel(xyz1, xyz2, flow1):
    B, C, N1 = xyz1.shape
    N2 = xyz2.shape[2]
    q2 = jnp.transpose(xyz2, (0, 2, 1))
    f1t = jnp.transpose(flow1, (0, 2, 1))

    out = pl.pallas_call(
        functools.partial(_warp_kernel, n1=N1),
        grid=(B, N2 // _BQ),
        in_specs=[
            pl.BlockSpec((1, C, N1), lambda b, i: (b, 0, 0)),
            pl.BlockSpec((1, _BQ, C), lambda b, i: (b, i, 0)),
            pl.BlockSpec((1, C, N1), lambda b, i: (b, 0, 0)),
            pl.BlockSpec((1, N1, C), lambda b, i: (b, 0, 0)),
        ],
        out_specs=pl.BlockSpec((1, _BQ, C), lambda b, i: (b, i, 0)),
        out_shape=jax.ShapeDtypeStruct((B, N2, C), jnp.float32),
        compiler_params=pltpu.CompilerParams(
            dimension_semantics=("parallel", "parallel")),
    )(xyz1, q2, flow1, f1t)
    return jnp.transpose(out, (0, 2, 1))


# speedup vs baseline: 1.3319x; 1.3319x over previous
import functools

import jax
import jax.numpy as jnp
from jax.experimental import pallas as pl
from jax.experimental.pallas import tpu as pltpu

_BQ = 256


def _warp_kernel(x1_ref, q2_ref, f1_ref, f1t_ref, out_ref, *, n1):
    x1 = x1_ref[0]
    f1 = f1_ref[0]
    f1t = f1t_ref[0]
    q = q2_ref[0]
    db = x1 + f1

    d = None
    for c in range(3):
        diff = q[:, c:c + 1] - db[c:c + 1, :]
        d = diff * diff if d is None else d + diff * diff

    iota = jax.lax.broadcasted_iota(jnp.int32, d.shape, 1)
    inf = jnp.float32(jnp.inf)

    invs = []
    dcur = d
    W = None
    for k in range(3):
        m = jnp.min(dcur, axis=1, keepdims=True)
        idx = jnp.argmin(dcur, axis=1, keepdims=True)
        oh = iota == idx
        inv = 1.0 / jnp.maximum(jnp.sqrt(m), 1e-10)
        invs.append(inv)
        W = jnp.where(oh, inv, 0.0) if k == 0 else jnp.where(oh, inv, W)
        if k < 2:
            dcur = jnp.where(oh, inf, dcur)

    rnorm = 1.0 / (invs[0] + invs[1] + invs[2])

    s = jax.lax.dot_general(W, f1t, (((1,), (0,)), ((), ())),
                            preferred_element_type=jnp.float32)
    out_ref[0] = q - s * rnorm


@jax.jit
def kernel(xyz1, xyz2, flow1):
    B, C, N1 = xyz1.shape
    N2 = xyz2.shape[2]
    q2 = jnp.transpose(xyz2, (0, 2, 1))
    f1t = jnp.transpose(flow1, (0, 2, 1))

    out = pl.pallas_call(
        functools.partial(_warp_kernel, n1=N1),
        grid=(B, N2 // _BQ),
        in_specs=[
            pl.BlockSpec((1, C, N1), lambda b, i: (b, 0, 0)),
            pl.BlockSpec((1, _BQ, C), lambda b, i: (b, i, 0)),
            pl.BlockSpec((1, C, N1), lambda b, i: (b, 0, 0)),
            pl.BlockSpec((1, N1, C), lambda b, i: (b, 0, 0)),
        ],
        out_specs=pl.BlockSpec((1, _BQ, C), lambda b, i: (b, i, 0)),
        out_shape=jax.ShapeDtypeStruct((B, N2, C), jnp.float32),
        compiler_params=pltpu.CompilerParams(
            dimension_semantics=("parallel", "parallel")),
    )(xyz1, q2, flow1, f1t)
    return jnp.transpose(out, (0, 2, 1))


# speedup vs baseline: 1.5613x; 1.1722x over previous
import functools

import jax
import jax.numpy as jnp
from jax.experimental import pallas as pl
from jax.experimental.pallas import tpu as pltpu

_BQ = 256


def _warp_kernel(x1_ref, q2_ref, f1_ref, f1t_ref, out_ref, *, n1):
    x1 = x1_ref[0]
    f1 = f1_ref[0]
    f1t = f1t_ref[0]
    q = q2_ref[0]
    db = x1 + f1

    d = None
    for c in range(3):
        diff = q[:, c:c + 1] - db[c:c + 1, :]
        d = diff * diff if d is None else d + diff * diff

    iota = jax.lax.broadcasted_iota(jnp.int32, d.shape, 1)
    inf = jnp.float32(jnp.inf)

    invs = []
    dcur = d
    W = None
    for k in range(3):
        m = jnp.min(dcur, axis=1, keepdims=True)
        idx = jnp.min(jnp.where(dcur <= m, iota, n1), axis=1,
                      keepdims=True)
        oh = iota == idx
        inv = 1.0 / jnp.maximum(jnp.sqrt(m), 1e-10)
        invs.append(inv)
        W = jnp.where(oh, inv, 0.0) if k == 0 else jnp.where(oh, inv, W)
        if k < 2:
            dcur = jnp.where(oh, inf, dcur)

    rnorm = 1.0 / (invs[0] + invs[1] + invs[2])

    s = jax.lax.dot_general(W, f1t, (((1,), (0,)), ((), ())),
                            preferred_element_type=jnp.float32)
    out_ref[0] = q - s * rnorm


@jax.jit
def kernel(xyz1, xyz2, flow1):
    B, C, N1 = xyz1.shape
    N2 = xyz2.shape[2]
    q2 = jnp.transpose(xyz2, (0, 2, 1))
    f1t = jnp.transpose(flow1, (0, 2, 1))

    out = pl.pallas_call(
        functools.partial(_warp_kernel, n1=N1),
        grid=(B, N2 // _BQ),
        in_specs=[
            pl.BlockSpec((1, C, N1), lambda b, i: (b, 0, 0)),
            pl.BlockSpec((1, _BQ, C), lambda b, i: (b, i, 0)),
            pl.BlockSpec((1, C, N1), lambda b, i: (b, 0, 0)),
            pl.BlockSpec((1, N1, C), lambda b, i: (b, 0, 0)),
        ],
        out_specs=pl.BlockSpec((1, _BQ, C), lambda b, i: (b, i, 0)),
        out_shape=jax.ShapeDtypeStruct((B, N2, C), jnp.float32),
        compiler_params=pltpu.CompilerParams(
            dimension_semantics=("parallel", "parallel")),
    )(xyz1, q2, flow1, f1t)
    return jnp.transpose(out, (0, 2, 1))


# speedup vs baseline: 1.6283x; 1.0429x over previous
import functools

import jax
import jax.numpy as jnp
from jax.experimental import pallas as pl
from jax.experimental.pallas import tpu as pltpu

_BQ = 256


def _warp_kernel(x1_ref, x2_ref, f1_ref, out_ref, *, n1):
    x1 = x1_ref[0]
    f1 = f1_ref[0]
    qs = x2_ref[0]
    db = x1 + f1
    q = qs.T

    d = None
    for c in range(3):
        diff = q[:, c:c + 1] - db[c:c + 1, :]
        d = diff * diff if d is None else d + diff * diff

    iota = jax.lax.broadcasted_iota(jnp.int32, d.shape, 1)
    inf = jnp.float32(jnp.inf)

    invs = []
    dcur = d
    W = None
    for k in range(3):
        m = jnp.min(dcur, axis=1, keepdims=True)
        idx = jnp.min(jnp.where(dcur <= m, iota, n1), axis=1,
                      keepdims=True)
        oh = iota == idx
        inv = 1.0 / jnp.maximum(jnp.sqrt(m), 1e-10)
        invs.append(inv)
        W = jnp.where(oh, inv, 0.0) if k == 0 else jnp.where(oh, inv, W)
        if k < 2:
            dcur = jnp.where(oh, inf, dcur)

    rnorm = 1.0 / (invs[0] + invs[1] + invs[2])

    s_t = jax.lax.dot_general(f1, W, (((1,), (1,)), ((), ())),
                              preferred_element_type=jnp.float32)
    out_ref[0] = qs - s_t * rnorm.T


@jax.jit
def kernel(xyz1, xyz2, flow1):
    B, C, N1 = xyz1.shape
    N2 = xyz2.shape[2]

    return pl.pallas_call(
        functools.partial(_warp_kernel, n1=N1),
        grid=(B, N2 // _BQ),
        in_specs=[
            pl.BlockSpec((1, C, N1), lambda b, i: (b, 0, 0)),
            pl.BlockSpec((1, C, _BQ), lambda b, i: (b, 0, i)),
            pl.BlockSpec((1, C, N1), lambda b, i: (b, 0, 0)),
        ],
        out_specs=pl.BlockSpec((1, C, _BQ), lambda b, i: (b, 0, i)),
        out_shape=jax.ShapeDtypeStruct((B, C, N2), jnp.float32),
        compiler_params=pltpu.CompilerParams(
            dimension_semantics=("parallel", "parallel")),
    )(xyz1, xyz2, flow1)


# speedup vs baseline: 1.6660x; 1.0232x over previous
import functools

import jax
import jax.numpy as jnp
from jax.experimental import pallas as pl
from jax.experimental.pallas import tpu as pltpu

_BQ = 512


def _warp_kernel(x1_ref, x2_ref, f1_ref, out_ref, *, n1):
    x1 = x1_ref[0]
    f1 = f1_ref[0]
    qs = x2_ref[0]
    db = x1 + f1
    q = qs.T

    d = None
    for c in range(3):
        diff = q[:, c:c + 1] - db[c:c + 1, :]
        d = diff * diff if d is None else d + diff * diff

    iota = jax.lax.broadcasted_iota(jnp.int32, d.shape, 1)
    inf = jnp.float32(jnp.inf)

    invs = []
    dcur = d
    W = None
    for k in range(3):
        m = jnp.min(dcur, axis=1, keepdims=True)
        idx = jnp.min(jnp.where(dcur <= m, iota, n1), axis=1,
                      keepdims=True)
        oh = iota == idx
        inv = 1.0 / jnp.maximum(jnp.sqrt(m), 1e-10)
        invs.append(inv)
        W = jnp.where(oh, inv, 0.0) if k == 0 else jnp.where(oh, inv, W)
        if k < 2:
            dcur = jnp.where(oh, inf, dcur)

    rnorm = 1.0 / (invs[0] + invs[1] + invs[2])

    s_t = jax.lax.dot_general(f1, W, (((1,), (1,)), ((), ())),
                              preferred_element_type=jnp.float32)
    out_ref[0] = qs - s_t * rnorm.T


@jax.jit
def kernel(xyz1, xyz2, flow1):
    B, C, N1 = xyz1.shape
    N2 = xyz2.shape[2]

    return pl.pallas_call(
        functools.partial(_warp_kernel, n1=N1),
        grid=(B, N2 // _BQ),
        in_specs=[
            pl.BlockSpec((1, C, N1), lambda b, i: (b, 0, 0)),
            pl.BlockSpec((1, C, _BQ), lambda b, i: (b, 0, i)),
            pl.BlockSpec((1, C, N1), lambda b, i: (b, 0, 0)),
        ],
        out_specs=pl.BlockSpec((1, C, _BQ), lambda b, i: (b, 0, i)),
        out_shape=jax.ShapeDtypeStruct((B, C, N2), jnp.float32),
        compiler_params=pltpu.CompilerParams(
            dimension_semantics=("parallel", "parallel")),
    )(xyz1, xyz2, flow1)
